# baseline (speedup 1.0000x reference)
import functools

import jax
import jax.numpy as jnp
from jax import lax
from jax.experimental import pallas as pl
from jax.experimental.pallas import tpu as pltpu
from jax.experimental.pallas import tpu_sc as plsc

N_NODES = 10000
N_EDGES = 320000
D_FEAT = 128
D_EDGE = 16

NW = 32
NSUB = 16
E_PER_W = N_EDGES // NW
EBLK = 80
NBLKW = E_PER_W // EBLK
NPAIR = (NBLKW - 1) // 2

C_ROWS = 3200
NX_ROWS = 1000


def _ab_body(x_ref, w1_ref, w2_ref, a_ref, b_ref):
    xv = x_ref[...]
    a_ref[...] = jnp.dot(xv, w1_ref[...], preferred_element_type=jnp.float32)
    b_ref[...] = jnp.dot(xv, w2_ref[...], preferred_element_type=jnp.float32)


def _ab_call(x, w1, w2):
    return pl.pallas_call(
        _ab_body,
        out_shape=[
            jax.ShapeDtypeStruct((N_NODES, D_EDGE), jnp.float32),
            jax.ShapeDtypeStruct((N_NODES, D_EDGE), jnp.float32),
        ],
    )(x, w1, w2)


E_PACK = N_EDGES // 8
C_ROWS128 = 4000


def _c_body(ea_ref, w8_ref, b_ref, c_ref):
    c_ref[...] = (
        jnp.dot(ea_ref[...], w8_ref[...], preferred_element_type=jnp.float32)
        + b_ref[...][0:1, :]
    )


def _c_call(ea40, w8, b8):
    grid = E_PACK // C_ROWS128
    return pl.pallas_call(
        _c_body,
        grid=(grid,),
        in_specs=[
            pl.BlockSpec((C_ROWS128, D_FEAT), lambda i: (i, 0)),
            pl.BlockSpec((D_FEAT, D_FEAT), lambda i: (0, 0)),
            pl.BlockSpec((8, D_FEAT), lambda i: (0, 0)),
        ],
        out_specs=pl.BlockSpec((C_ROWS128, D_FEAT), lambda i: (i, 0)),
        out_shape=jax.ShapeDtypeStruct((E_PACK, D_FEAT), jnp.float32),
    )(ea40, w8, b8)


CHUNK = 400
NCHUNK = N_NODES // CHUNK


CBLK = EBLK // 8


def _sc_body(row_hbm, col_hbm, a_hbm, b_hbm, c_hbm,
             eout_hbm, agg_hbm,
             ri_0, ri_1, ri_2, ri_3,
             ci_0, ci_1, ci_2, ci_3,
             av_0, av_1, bv_0, bv_1, cv_0, cv_1,
             ov_0, ov_1, op_0, op_1, z_v,
             agg_sh,
             sem_i0, sem_i1, sem_i2, sem_i3,
             sem_g0, sem_g1, sem_w0, sem_w1, sem_s0, sem_s1):
    cid = lax.axis_index("c")
    sid = lax.axis_index("s")
    wid = sid * 2 + cid
    base_w = wid * E_PER_W
    base_p = wid * (E_PER_W // 8)
    ri = (ri_0, ri_1, ri_2, ri_3)
    ci = (ci_0, ci_1, ci_2, ci_3)
    av = (av_0, av_1)
    bv = (bv_0, bv_1)
    cv = (cv_0, cv_1)
    ov = (ov_0, ov_1)
    op = (op_0, op_1)
    sem_i = (sem_i0, sem_i1, sem_i2, sem_i3)
    sem_g = (sem_g0, sem_g1)
    sem_w = (sem_w0, sem_w1)
    sem_s = (sem_s0, sem_s1)

    zero_row = jnp.zeros((D_EDGE,), jnp.float32)

    def _zero(j, carry):
        z_v[j, :] = zero_row
        return carry

    lax.fori_loop(0, CHUNK, _zero, 0)

    def _prep(j):
        pltpu.sync_copy(z_v, agg_sh.at[pl.ds(j * CHUNK, CHUNK)])

    _prep(sid)

    @pl.when(sid + NSUB < NCHUNK)
    def _():
        _prep(sid + NSUB)

    plsc.subcore_barrier()

    def idx_load(i, si):
        b = base_w + i * EBLK
        pltpu.async_copy(row_hbm.at[pl.ds(b, EBLK)], ri[si], sem_i[si])
        pltpu.async_copy(col_hbm.at[pl.ds(b, EBLK)], ci[si], sem_i[si])

    def idx_wait(si):
        pltpu.make_async_copy(row_hbm.at[pl.ds(0, EBLK)], ri[si],
                              sem_i[si]).wait()
        pltpu.make_async_copy(col_hbm.at[pl.ds(0, EBLK)], ci[si],
                              sem_i[si]).wait()

    def gather_issue(i, sd, si):
        pltpu.async_copy(a_hbm.at[ri[si]], av[sd], sem_g[sd])
        pltpu.async_copy(b_hbm.at[ci[si]], bv[sd], sem_g[sd])
        pltpu.async_copy(c_hbm.at[pl.ds(base_p + i * CBLK, CBLK)],
                         cv[sd], sem_g[sd])

    def gather_wait(sd):
        dummy = a_hbm.at[pl.ds(0, EBLK)]
        pltpu.make_async_copy(dummy, av[sd], sem_g[sd]).wait()
        pltpu.make_async_copy(dummy, bv[sd], sem_g[sd]).wait()
        pltpu.make_async_copy(c_hbm.at[pl.ds(0, CBLK)], cv[sd],
                              sem_g[sd]).wait()

    def write_issue(i, sd, si):
        pltpu.async_copy(op[sd], eout_hbm.at[pl.ds(base_p + i * CBLK, CBLK)],
                         sem_w[sd])
        pltpu.async_copy(ov[sd], agg_sh.at[ci[si]], sem_s[sd], add=True)

    def write_wait(sd, si):
        pltpu.make_async_copy(op[sd], eout_hbm.at[pl.ds(0, CBLK)],
                              sem_w[sd]).wait()
        pltpu.make_async_copy(ov[sd], agg_sh.at[ci[si]], sem_s[sd]).wait()

    def compute(sd):
        for j in range(CBLK):
            for u in range(8):
                r = j * 8 + u
                sl = pl.ds(u * D_EDGE, D_EDGE)
                val = jnp.maximum(
                    av[sd][r, :] + bv[sd][r, :] + cv[sd][j, sl], 0.0)
                ov[sd][r, :] = val
                op[sd][j, sl] = val

    def blk_step(i, u):
        sd, so = u % 2, (u + 1) % 2
        idx_wait((u + 1) % 4)

        @pl.when(i >= 1)
        def _():
            write_wait(so, (u + 3) % 4)

        gather_issue(i + 1, so, (u + 1) % 4)
        gather_wait(sd)

        @pl.when(i + 2 < NBLKW)
        def _():
            idx_load(i + 2, (u + 2) % 4)

        compute(sd)
        write_issue(i, sd, u)

    idx_load(0, 0)
    idx_load(1, 1)
    idx_wait(0)
    gather_issue(0, 0, 0)

    def _quad(k, carry):
        for u in range(4):
            blk_step(4 * k + u, u)
        return carry

    lax.fori_loop(0, NBLKW // 4, _quad, 0)

    last = NBLKW - 1
    gather_wait(0)
    compute(0)
    write_issue(last, 0, 0)
    write_wait(1, 3)
    write_wait(0, 0)
    plsc.subcore_barrier()

    def _drain(j):
        pltpu.sync_copy(agg_sh.at[pl.ds(j * CHUNK, CHUNK)], z_v)
        pltpu.sync_copy(z_v, agg_hbm.at[cid, j])

    _drain(sid)

    @pl.when(sid + NSUB < NCHUNK)
    def _():
        _drain(sid + NSUB)


def _sc_call(row, col, a, b, c):
    mesh = plsc.VectorSubcoreMesh(core_axis_name="c", subcore_axis_name="s")
    fn = pl.kernel(
        _sc_body,
        out_type=[
            jax.ShapeDtypeStruct((E_PACK, D_FEAT), jnp.float32),
            jax.ShapeDtypeStruct((2, NCHUNK, CHUNK, D_EDGE), jnp.float32),
        ],
        mesh=mesh,
        scratch_types=(
            [pltpu.VMEM((EBLK,), jnp.int32)] * 8
            + [pltpu.VMEM((EBLK, D_EDGE), jnp.float32)] * 4
            + [pltpu.VMEM((CBLK, D_FEAT), jnp.float32)] * 2
            + [pltpu.VMEM((EBLK, D_EDGE), jnp.float32)] * 2
            + [pltpu.VMEM((CBLK, D_FEAT), jnp.float32)] * 2
            + [pltpu.VMEM((CHUNK, D_EDGE), jnp.float32),
               pltpu.VMEM_SHARED((N_NODES, D_EDGE), jnp.float32)]
            + [pltpu.SemaphoreType.DMA] * 10
        ),
        compiler_params=pltpu.CompilerParams(use_tc_tiling_on_sc=False),
    )
    return fn(row, col, a, b, c)


def _nx_body(x_ref, agg_ref, w1_ref, w2_ref, b_ref, o_ref):
    aggs = agg_ref[0, 0] + agg_ref[1, 0]
    o_ref[...] = (
        jnp.dot(x_ref[...], w1_ref[...], preferred_element_type=jnp.float32)
        + jnp.dot(aggs, w2_ref[...], preferred_element_type=jnp.float32)
        + b_ref[...][0:1, :]
    )


def _nx_call(x, agg4, w1, w2, b8):
    return pl.pallas_call(
        _nx_body,
        grid=(NCHUNK,),
        in_specs=[
            pl.BlockSpec((CHUNK, D_FEAT), lambda i: (i, 0)),
            pl.BlockSpec((2, 1, CHUNK, D_EDGE), lambda i: (0, i, 0, 0)),
            pl.BlockSpec((D_FEAT, D_FEAT), lambda i: (0, 0)),
            pl.BlockSpec((D_EDGE, D_FEAT), lambda i: (0, 0)),
            pl.BlockSpec((8, D_FEAT), lambda i: (0, 0)),
        ],
        out_specs=pl.BlockSpec((CHUNK, D_FEAT), lambda i: (i, 0)),
        out_shape=jax.ShapeDtypeStruct((N_NODES, D_FEAT), jnp.float32),
    )(x, agg4, w1, w2, b8)


@jax.jit
def kernel(x, edge_index, edge_attr, We_w, We_b, Wn_w, Wn_b):
    ei = edge_index.astype(jnp.int32)
    row = ei[0]
    col = ei[1]
    a, b = _ab_call(x, We_w[:D_FEAT], We_w[D_FEAT:2 * D_FEAT])
    w8 = jnp.kron(jnp.eye(8, dtype=jnp.float32), We_w[2 * D_FEAT:])
    b128 = jnp.tile(We_b, 8)
    c40 = _c_call(edge_attr.reshape(E_PACK, D_FEAT), w8,
                  jnp.broadcast_to(b128.reshape(1, D_FEAT), (8, D_FEAT)))
    eout40, agg4 = _sc_call(row, col, a, b, c40)
    new_edge_attr = eout40.reshape(N_EDGES, D_EDGE)
    new_x = _nx_call(x, agg4, Wn_w[:D_FEAT], Wn_w[D_FEAT:],
                     jnp.broadcast_to(Wn_b.reshape(1, D_FEAT), (8, D_FEAT)))
    return new_x, new_edge_attr


# speedup vs baseline: 1.0180x; 1.0180x over previous
import functools

import jax
import jax.numpy as jnp
from jax import lax
from jax.experimental import pallas as pl
from jax.experimental.pallas import tpu as pltpu
from jax.experimental.pallas import tpu_sc as plsc

N_NODES = 10000
N_EDGES = 320000
D_FEAT = 128
D_EDGE = 16

NW = 32
NSUB = 16
E_PER_W = N_EDGES // NW
EBLK = 80
NBLKW = E_PER_W // EBLK
NPAIR = (NBLKW - 1) // 2

C_ROWS = 3200
NX_ROWS = 1000


E_PACK = N_EDGES // 8
C_ROWS128 = 4000


def _c_body(x_ref, ea_ref, w1_ref, w2_ref, w8_ref, b_ref,
            a_ref, bt_ref, c_ref):
    @pl.when(pl.program_id(0) == 0)
    def _():
        xv = x_ref[...]
        a_ref[...] = jnp.dot(xv, w1_ref[...],
                             preferred_element_type=jnp.float32)
        bt_ref[...] = jnp.dot(xv, w2_ref[...],
                              preferred_element_type=jnp.float32)

    c_ref[...] = (
        jnp.dot(ea_ref[...], w8_ref[...], preferred_element_type=jnp.float32)
        + b_ref[...][0:1, :]
    )


def _c_call(x, ea40, w1, w2, w8, b8):
    grid = E_PACK // C_ROWS128
    return pl.pallas_call(
        _c_body,
        grid=(grid,),
        in_specs=[
            pl.BlockSpec((N_NODES, D_FEAT), lambda i: (0, 0)),
            pl.BlockSpec((C_ROWS128, D_FEAT), lambda i: (i, 0)),
            pl.BlockSpec((D_FEAT, D_EDGE), lambda i: (0, 0)),
            pl.BlockSpec((D_FEAT, D_EDGE), lambda i: (0, 0)),
            pl.BlockSpec((D_FEAT, D_FEAT), lambda i: (0, 0)),
            pl.BlockSpec((8, D_FEAT), lambda i: (0, 0)),
        ],
        out_specs=[
            pl.BlockSpec((N_NODES, D_EDGE), lambda i: (0, 0)),
            pl.BlockSpec((N_NODES, D_EDGE), lambda i: (0, 0)),
            pl.BlockSpec((C_ROWS128, D_FEAT), lambda i: (i, 0)),
        ],
        out_shape=[
            jax.ShapeDtypeStruct((N_NODES, D_EDGE), jnp.float32),
            jax.ShapeDtypeStruct((N_NODES, D_EDGE), jnp.float32),
            jax.ShapeDtypeStruct((E_PACK, D_FEAT), jnp.float32),
        ],
    )(x, ea40, w1, w2, w8, b8)


CHUNK = 400
NCHUNK = N_NODES // CHUNK


CBLK = EBLK // 8


def _sc_body(row_hbm, col_hbm, a_hbm, b_hbm, c_hbm,
             eout_hbm, agg_hbm,
             ri_0, ri_1, ri_2, ri_3,
             ci_0, ci_1, ci_2, ci_3,
             av_0, av_1, bv_0, bv_1, cv_0, cv_1,
             ov_0, ov_1, op_0, op_1, z_v,
             agg_sh,
             sem_i0, sem_i1, sem_i2, sem_i3,
             sem_g0, sem_g1, sem_w0, sem_w1, sem_s0, sem_s1):
    cid = lax.axis_index("c")
    sid = lax.axis_index("s")
    wid = sid * 2 + cid
    base_w = wid * E_PER_W
    base_p = wid * (E_PER_W // 8)
    ri = (ri_0, ri_1, ri_2, ri_3)
    ci = (ci_0, ci_1, ci_2, ci_3)
    av = (av_0, av_1)
    bv = (bv_0, bv_1)
    cv = (cv_0, cv_1)
    ov = (ov_0, ov_1)
    op = (op_0, op_1)
    sem_i = (sem_i0, sem_i1, sem_i2, sem_i3)
    sem_g = (sem_g0, sem_g1)
    sem_w = (sem_w0, sem_w1)
    sem_s = (sem_s0, sem_s1)

    zero_row = jnp.zeros((D_EDGE,), jnp.float32)

    def _zero(j, carry):
        z_v[j, :] = zero_row
        return carry

    lax.fori_loop(0, CHUNK, _zero, 0)

    def _prep(j):
        pltpu.sync_copy(z_v, agg_sh.at[pl.ds(j * CHUNK, CHUNK)])

    _prep(sid)

    @pl.when(sid + NSUB < NCHUNK)
    def _():
        _prep(sid + NSUB)

    plsc.subcore_barrier()

    def idx_load(i, si):
        b = base_w + i * EBLK
        pltpu.async_copy(row_hbm.at[pl.ds(b, EBLK)], ri[si], sem_i[si])
        pltpu.async_copy(col_hbm.at[pl.ds(b, EBLK)], ci[si], sem_i[si])

    def idx_wait(si):
        pltpu.make_async_copy(row_hbm.at[pl.ds(0, EBLK)], ri[si],
                              sem_i[si]).wait()
        pltpu.make_async_copy(col_hbm.at[pl.ds(0, EBLK)], ci[si],
                              sem_i[si]).wait()

    def gather_issue(i, sd, si):
        pltpu.async_copy(a_hbm.at[ri[si]], av[sd], sem_g[sd])
        pltpu.async_copy(b_hbm.at[ci[si]], bv[sd], sem_g[sd])
        pltpu.async_copy(c_hbm.at[pl.ds(base_p + i * CBLK, CBLK)],
                         cv[sd], sem_g[sd])

    def gather_wait(sd):
        dummy = a_hbm.at[pl.ds(0, EBLK)]
        pltpu.make_async_copy(dummy, av[sd], sem_g[sd]).wait()
        pltpu.make_async_copy(dummy, bv[sd], sem_g[sd]).wait()
        pltpu.make_async_copy(c_hbm.at[pl.ds(0, CBLK)], cv[sd],
                              sem_g[sd]).wait()

    def write_issue(i, sd, si):
        pltpu.async_copy(op[sd], eout_hbm.at[pl.ds(base_p + i * CBLK, CBLK)],
                         sem_w[sd])
        pltpu.async_copy(ov[sd], agg_sh.at[ci[si]], sem_s[sd], add=True)

    def write_wait(sd, si):
        pltpu.make_async_copy(op[sd], eout_hbm.at[pl.ds(0, CBLK)],
                              sem_w[sd]).wait()
        pltpu.make_async_copy(ov[sd], agg_sh.at[ci[si]], sem_s[sd]).wait()

    def compute(sd):
        for j in range(CBLK):
            for u in range(8):
                r = j * 8 + u
                sl = pl.ds(u * D_EDGE, D_EDGE)
                val = jnp.maximum(
                    av[sd][r, :] + bv[sd][r, :] + cv[sd][j, sl], 0.0)
                ov[sd][r, :] = val
                op[sd][j, sl] = val

    def blk_step(i, u):
        sd, so = u % 2, (u + 1) % 2
        idx_wait((u + 1) % 4)

        @pl.when(i >= 1)
        def _():
            write_wait(so, (u + 3) % 4)

        gather_issue(i + 1, so, (u + 1) % 4)
        gather_wait(sd)

        @pl.when(i + 2 < NBLKW)
        def _():
            idx_load(i + 2, (u + 2) % 4)

        compute(sd)
        write_issue(i, sd, u)

    idx_load(0, 0)
    idx_load(1, 1)
    idx_wait(0)
    gather_issue(0, 0, 0)

    def _quad(k, carry):
        for u in range(4):
            blk_step(4 * k + u, u)
        return carry

    lax.fori_loop(0, NBLKW // 4, _quad, 0)

    last = NBLKW - 1
    gather_wait(0)
    compute(0)
    write_issue(last, 0, 0)
    write_wait(1, 3)
    write_wait(0, 0)
    plsc.subcore_barrier()

    def _drain(j):
        pltpu.sync_copy(agg_sh.at[pl.ds(j * CHUNK, CHUNK)], z_v)
        pltpu.sync_copy(z_v, agg_hbm.at[cid, j])

    _drain(sid)

    @pl.when(sid + NSUB < NCHUNK)
    def _():
        _drain(sid + NSUB)


def _sc_call(row, col, a, b, c):
    mesh = plsc.VectorSubcoreMesh(core_axis_name="c", subcore_axis_name="s")
    fn = pl.kernel(
        _sc_body,
        out_type=[
            jax.ShapeDtypeStruct((E_PACK, D_FEAT), jnp.float32),
            jax.ShapeDtypeStruct((2, NCHUNK, CHUNK, D_EDGE), jnp.float32),
        ],
        mesh=mesh,
        scratch_types=(
            [pltpu.VMEM((EBLK,), jnp.int32)] * 8
            + [pltpu.VMEM((EBLK, D_EDGE), jnp.float32)] * 4
            + [pltpu.VMEM((CBLK, D_FEAT), jnp.float32)] * 2
            + [pltpu.VMEM((EBLK, D_EDGE), jnp.float32)] * 2
            + [pltpu.VMEM((CBLK, D_FEAT), jnp.float32)] * 2
            + [pltpu.VMEM((CHUNK, D_EDGE), jnp.float32),
               pltpu.VMEM_SHARED((N_NODES, D_EDGE), jnp.float32)]
            + [pltpu.SemaphoreType.DMA] * 10
        ),
        compiler_params=pltpu.CompilerParams(use_tc_tiling_on_sc=False),
    )
    return fn(row, col, a, b, c)


def _nx_body(x_ref, agg_ref, w1_ref, w2_ref, b_ref, o_ref):
    aggs = agg_ref[0, 0] + agg_ref[1, 0]
    o_ref[...] = (
        jnp.dot(x_ref[...], w1_ref[...], preferred_element_type=jnp.float32)
        + jnp.dot(aggs, w2_ref[...], preferred_element_type=jnp.float32)
        + b_ref[...][0:1, :]
    )


def _nx_call(x, agg4, w1, w2, b8):
    return pl.pallas_call(
        _nx_body,
        grid=(NCHUNK,),
        in_specs=[
            pl.BlockSpec((CHUNK, D_FEAT), lambda i: (i, 0)),
            pl.BlockSpec((2, 1, CHUNK, D_EDGE), lambda i: (0, i, 0, 0)),
            pl.BlockSpec((D_FEAT, D_FEAT), lambda i: (0, 0)),
            pl.BlockSpec((D_EDGE, D_FEAT), lambda i: (0, 0)),
            pl.BlockSpec((8, D_FEAT), lambda i: (0, 0)),
        ],
        out_specs=pl.BlockSpec((CHUNK, D_FEAT), lambda i: (i, 0)),
        out_shape=jax.ShapeDtypeStruct((N_NODES, D_FEAT), jnp.float32),
    )(x, agg4, w1, w2, b8)


@jax.jit
def kernel(x, edge_index, edge_attr, We_w, We_b, Wn_w, Wn_b):
    ei = edge_index.astype(jnp.int32)
    row = ei[0]
    col = ei[1]
    w8 = jnp.kron(jnp.eye(8, dtype=jnp.float32), We_w[2 * D_FEAT:])
    b128 = jnp.tile(We_b, 8)
    a, b, c40 = _c_call(x, edge_attr.reshape(E_PACK, D_FEAT),
                        We_w[:D_FEAT], We_w[D_FEAT:2 * D_FEAT], w8,
                        jnp.broadcast_to(b128.reshape(1, D_FEAT), (8, D_FEAT)))
    eout40, agg4 = _sc_call(row, col, a, b, c40)
    new_edge_attr = eout40.reshape(N_EDGES, D_EDGE)
    new_x = _nx_call(x, agg4, Wn_w[:D_FEAT], Wn_w[D_FEAT:],
                     jnp.broadcast_to(Wn_b.reshape(1, D_FEAT), (8, D_FEAT)))
    return new_x, new_edge_attr


# speedup vs baseline: 1.0181x; 1.0001x over previous
import functools

import jax
import jax.numpy as jnp
from jax import lax
from jax.experimental import pallas as pl
from jax.experimental.pallas import tpu as pltpu
from jax.experimental.pallas import tpu_sc as plsc

N_NODES = 10000
N_EDGES = 320000
D_FEAT = 128
D_EDGE = 16

NW = 32
NSUB = 16
E_PER_W = N_EDGES // NW
EBLK = 80
NBLKW = E_PER_W // EBLK


E_PACK = N_EDGES // 8
C_ROWS128 = 4000


def _c_body(x_ref, ea_ref, w1_ref, w2_ref, w8_ref, b_ref,
            a_ref, bt_ref, c_ref):
    @pl.when(pl.program_id(0) == 0)
    def _():
        xv = x_ref[...]
        a_ref[...] = jnp.dot(xv, w1_ref[...],
                             preferred_element_type=jnp.float32)
        bt_ref[...] = jnp.dot(xv, w2_ref[...],
                              preferred_element_type=jnp.float32)

    c_ref[...] = (
        jnp.dot(ea_ref[...], w8_ref[...], preferred_element_type=jnp.float32)
        + b_ref[...][0:1, :]
    )


def _c_call(x, ea40, w1, w2, w8, b8):
    grid = E_PACK // C_ROWS128
    return pl.pallas_call(
        _c_body,
        grid=(grid,),
        in_specs=[
            pl.BlockSpec((N_NODES, D_FEAT), lambda i: (0, 0)),
            pl.BlockSpec((C_ROWS128, D_FEAT), lambda i: (i, 0)),
            pl.BlockSpec((D_FEAT, D_EDGE), lambda i: (0, 0)),
            pl.BlockSpec((D_FEAT, D_EDGE), lambda i: (0, 0)),
            pl.BlockSpec((D_FEAT, D_FEAT), lambda i: (0, 0)),
            pl.BlockSpec((8, D_FEAT), lambda i: (0, 0)),
        ],
        out_specs=[
            pl.BlockSpec((N_NODES, D_EDGE), lambda i: (0, 0)),
            pl.BlockSpec((N_NODES, D_EDGE), lambda i: (0, 0)),
            pl.BlockSpec((C_ROWS128, D_FEAT), lambda i: (i, 0)),
        ],
        out_shape=[
            jax.ShapeDtypeStruct((N_NODES, D_EDGE), jnp.float32),
            jax.ShapeDtypeStruct((N_NODES, D_EDGE), jnp.float32),
            jax.ShapeDtypeStruct((E_PACK, D_FEAT), jnp.float32),
        ],
    )(x, ea40, w1, w2, w8, b8)


CHUNK = 400
NCHUNK = N_NODES // CHUNK


CBLK = EBLK // 8


def _sc_body(row_hbm, col_hbm, a_hbm, b_hbm, c_hbm,
             eout_hbm, agg_hbm,
             ri_0, ri_1, ri_2, ri_3,
             ci_0, ci_1, ci_2, ci_3,
             av_0, av_1, bv_0, bv_1, cv_0, cv_1,
             ov_0, ov_1, op_0, op_1, z_v,
             agg_sh,
             sem_i0, sem_i1, sem_i2, sem_i3,
             sem_g0, sem_g1, sem_w0, sem_w1, sem_s0, sem_s1):
    cid = lax.axis_index("c")
    sid = lax.axis_index("s")
    wid = sid * 2 + cid
    base_w = wid * E_PER_W
    base_p = wid * (E_PER_W // 8)
    ri = (ri_0, ri_1, ri_2, ri_3)
    ci = (ci_0, ci_1, ci_2, ci_3)
    av = (av_0, av_1)
    bv = (bv_0, bv_1)
    cv = (cv_0, cv_1)
    ov = (ov_0, ov_1)
    op = (op_0, op_1)
    sem_i = (sem_i0, sem_i1, sem_i2, sem_i3)
    sem_g = (sem_g0, sem_g1)
    sem_w = (sem_w0, sem_w1)
    sem_s = (sem_s0, sem_s1)

    zero_row = jnp.zeros((D_EDGE,), jnp.float32)

    def _zero(j, carry):
        z_v[j, :] = zero_row
        return carry

    lax.fori_loop(0, CHUNK, _zero, 0)

    def _prep(j):
        pltpu.sync_copy(z_v, agg_sh.at[pl.ds(j * CHUNK, CHUNK)])

    _prep(sid)

    @pl.when(sid + NSUB < NCHUNK)
    def _():
        _prep(sid + NSUB)

    plsc.subcore_barrier()

    def idx_load(i, si):
        b = base_w + i * EBLK
        pltpu.async_copy(row_hbm.at[pl.ds(b, EBLK)], ri[si], sem_i[si])
        pltpu.async_copy(col_hbm.at[pl.ds(b, EBLK)], ci[si], sem_i[si])

    def idx_wait(si):
        pltpu.make_async_copy(row_hbm.at[pl.ds(0, EBLK)], ri[si],
                              sem_i[si]).wait()
        pltpu.make_async_copy(col_hbm.at[pl.ds(0, EBLK)], ci[si],
                              sem_i[si]).wait()

    def gather_issue(i, sd, si):
        pltpu.async_copy(a_hbm.at[ri[si]], av[sd], sem_g[sd])
        pltpu.async_copy(b_hbm.at[ci[si]], bv[sd], sem_g[sd])
        pltpu.async_copy(c_hbm.at[pl.ds(base_p + i * CBLK, CBLK)],
                         cv[sd], sem_g[sd])

    def gather_wait(sd):
        dummy = a_hbm.at[pl.ds(0, EBLK)]
        pltpu.make_async_copy(dummy, av[sd], sem_g[sd]).wait()
        pltpu.make_async_copy(dummy, bv[sd], sem_g[sd]).wait()
        pltpu.make_async_copy(c_hbm.at[pl.ds(0, CBLK)], cv[sd],
                              sem_g[sd]).wait()

    def write_issue(i, sd, si):
        pltpu.async_copy(op[sd], eout_hbm.at[pl.ds(base_p + i * CBLK, CBLK)],
                         sem_w[sd])
        pltpu.async_copy(ov[sd], agg_sh.at[ci[si]], sem_s[sd], add=True)

    def write_wait(sd, si):
        pltpu.make_async_copy(op[sd], eout_hbm.at[pl.ds(0, CBLK)],
                              sem_w[sd]).wait()
        pltpu.make_async_copy(ov[sd], agg_sh.at[ci[si]], sem_s[sd]).wait()

    def compute(sd):
        for j in range(CBLK):
            for u in range(8):
                r = j * 8 + u
                sl = pl.ds(u * D_EDGE, D_EDGE)
                val = jnp.maximum(
                    av[sd][r, :] + bv[sd][r, :] + cv[sd][j, sl], 0.0)
                ov[sd][r, :] = val
                op[sd][j, sl] = val

    def blk_step(i, u):
        sd, so = u % 2, (u + 1) % 2
        idx_wait((u + 1) % 4)

        @pl.when(i >= 1)
        def _():
            write_wait(so, (u + 3) % 4)

        gather_issue(i + 1, so, (u + 1) % 4)
        gather_wait(sd)

        @pl.when(i + 2 < NBLKW)
        def _():
            idx_load(i + 2, (u + 2) % 4)

        compute(sd)
        write_issue(i, sd, u)

    idx_load(0, 0)
    idx_load(1, 1)
    idx_wait(0)
    gather_issue(0, 0, 0)

    def _quad(k, carry):
        for u in range(4):
            blk_step(4 * k + u, u)
        return carry

    lax.fori_loop(0, NBLKW // 4, _quad, 0)

    last = NBLKW - 1
    gather_wait(0)
    compute(0)
    write_issue(last, 0, 0)
    write_wait(1, 3)
    write_wait(0, 0)
    plsc.subcore_barrier()

    def _drain(j):
        pltpu.sync_copy(agg_sh.at[pl.ds(j * CHUNK, CHUNK)], z_v)
        pltpu.sync_copy(z_v, agg_hbm.at[cid, j])

    _drain(sid)

    @pl.when(sid + NSUB < NCHUNK)
    def _():
        _drain(sid + NSUB)


def _sc_call(row, col, a, b, c):
    mesh = plsc.VectorSubcoreMesh(core_axis_name="c", subcore_axis_name="s")
    fn = pl.kernel(
        _sc_body,
        out_type=[
            jax.ShapeDtypeStruct((E_PACK, D_FEAT), jnp.float32),
            jax.ShapeDtypeStruct((2, NCHUNK, CHUNK, D_EDGE), jnp.float32),
        ],
        mesh=mesh,
        scratch_types=(
            [pltpu.VMEM((EBLK,), jnp.int32)] * 8
            + [pltpu.VMEM((EBLK, D_EDGE), jnp.float32)] * 4
            + [pltpu.VMEM((CBLK, D_FEAT), jnp.float32)] * 2
            + [pltpu.VMEM((EBLK, D_EDGE), jnp.float32)] * 2
            + [pltpu.VMEM((CBLK, D_FEAT), jnp.float32)] * 2
            + [pltpu.VMEM((CHUNK, D_EDGE), jnp.float32),
               pltpu.VMEM_SHARED((N_NODES, D_EDGE), jnp.float32)]
            + [pltpu.SemaphoreType.DMA] * 10
        ),
        compiler_params=pltpu.CompilerParams(use_tc_tiling_on_sc=False),
    )
    return fn(row, col, a, b, c)


def _nx_body(x_ref, agg_ref, w1_ref, w2_ref, b_ref, o_ref):
    aggs = agg_ref[0, 0] + agg_ref[1, 0]
    o_ref[...] = (
        jnp.dot(x_ref[...], w1_ref[...], preferred_element_type=jnp.float32)
        + jnp.dot(aggs, w2_ref[...], preferred_element_type=jnp.float32)
        + b_ref[...][0:1, :]
    )


def _nx_call(x, agg4, w1, w2, b8):
    return pl.pallas_call(
        _nx_body,
        grid=(NCHUNK,),
        in_specs=[
            pl.BlockSpec((CHUNK, D_FEAT), lambda i: (i, 0)),
            pl.BlockSpec((2, 1, CHUNK, D_EDGE), lambda i: (0, i, 0, 0)),
            pl.BlockSpec((D_FEAT, D_FEAT), lambda i: (0, 0)),
            pl.BlockSpec((D_EDGE, D_FEAT), lambda i: (0, 0)),
            pl.BlockSpec((8, D_FEAT), lambda i: (0, 0)),
        ],
        out_specs=pl.BlockSpec((CHUNK, D_FEAT), lambda i: (i, 0)),
        out_shape=jax.ShapeDtypeStruct((N_NODES, D_FEAT), jnp.float32),
    )(x, agg4, w1, w2, b8)


@jax.jit
def kernel(x, edge_index, edge_attr, We_w, We_b, Wn_w, Wn_b):
    ei = edge_index.astype(jnp.int32)
    row = ei[0]
    col = ei[1]
    w8 = jnp.kron(jnp.eye(8, dtype=jnp.float32), We_w[2 * D_FEAT:])
    b128 = jnp.tile(We_b, 8)
    a, b, c40 = _c_call(x, edge_attr.reshape(E_PACK, D_FEAT),
                        We_w[:D_FEAT], We_w[D_FEAT:2 * D_FEAT], w8,
                        jnp.broadcast_to(b128.reshape(1, D_FEAT), (8, D_FEAT)))
    eout40, agg4 = _sc_call(row, col, a, b, c40)
    new_edge_attr = eout40.reshape(N_EDGES, D_EDGE)
    new_x = _nx_call(x, agg4, Wn_w[:D_FEAT], Wn_w[D_FEAT:],
                     jnp.broadcast_to(Wn_b.reshape(1, D_FEAT), (8, D_FEAT)))
    return new_x, new_edge_attr
